# baseline (speedup 1.0000x reference)
import jax
import jax.numpy as jnp
from jax.experimental import pallas as pl


def _identity_kernel(x_ref, o_ref):
    o_ref[...] = x_ref[...]


def _gcn(x, ei, W, b):
    n = x.shape[0]
    sl = jnp.arange(n, dtype=ei.dtype)
    src = jnp.concatenate([ei[0], sl])
    dst = jnp.concatenate([ei[1], sl])
    deg = jnp.zeros((n,), jnp.float32).at[dst].add(1.0)
    dinv = 1.0 / jnp.sqrt(jnp.maximum(deg, 1.0))
    h = x @ W
    msg = h[src] * (dinv[src] * dinv[dst])[:, None]
    out = jnp.zeros_like(h).at[dst].add(msg)
    return out + b


def kernel(x_drug, edge_index_drug, batch_drug, x_prot, edge_index_prot, batch_prot, g1Wa, g1ba, g1Wb, g1bb, bn1g, bn1b, g2Wa, g2ba, g2Wb, g2bb, bn2g, bn2b, g3Wa, g3ba, g3Wb, g3bb, bn3g, bn3b, g4Wa, g4ba, g4Wb, g4bb, bn4g, bn4b, g5Wa, g5ba, g5Wb, g5bb, bn5g, bn5b, fxdW, fxdb, gc1W, gc1b, gc2W, gc2b, l1W, l1b, f1W, f1b, f2W, f2b, oW, ob):
    d = dict(locals())
    G = 512
    x = x_drug
    ei = edge_index_drug
    for i in range(1, 6):
        agg = jnp.zeros_like(x).at[ei[1]].add(x[ei[0]])
        h = x + agg
        h = jnp.maximum(h @ d['g%dWa' % i] + d['g%dba' % i], 0.0)
        h = h @ d['g%dWb' % i] + d['g%dbb' % i]
        h = jnp.maximum(h, 0.0)
        mu = h.mean(0)
        var = jnp.mean((h - mu) ** 2, 0)
        x = (h - mu) / jnp.sqrt(var + 1e-5) * d['bn%dg' % i] + d['bn%db' % i]
    xd = jax.ops.segment_sum(x, batch_drug, num_segments=G)
    xd = jnp.maximum(xd @ fxdW + fxdb, 0.0)
    xp = jnp.maximum(_gcn(x_prot, edge_index_prot, gc1W, gc1b), 0.0)
    xp = jnp.maximum(_gcn(xp, edge_index_prot, gc2W, gc2b), 0.0)
    xp = jax.ops.segment_max(xp, batch_prot, num_segments=G)
    xp = jnp.where(jnp.isfinite(xp), xp, 0.0)
    xp = jnp.maximum(xp @ l1W + l1b, 0.0)
    xc = jnp.concatenate([xd, xp], axis=1)
    xc = jnp.maximum(xc @ f1W + f1b, 0.0)
    xc = jnp.maximum(xc @ f2W + f2b, 0.0)
    out = xc @ oW + ob
    return pl.pallas_call(
        _identity_kernel,
        out_shape=jax.ShapeDtypeStruct(out.shape, out.dtype),
    )(out)


# speedup vs baseline: 10.2967x; 10.2967x over previous
import functools

import jax
import jax.numpy as jnp
from jax import lax
from jax.experimental import pallas as pl
from jax.experimental.pallas import tpu as pltpu
from jax.experimental.pallas import tpu_sc as plsc

_EC = 128
_NTILES = 32
_W = 32
_BR = 2000
_G = 512


def _scatter_body(n_nodes, n_edges, gather_rows, vals, src, dst, out,
                  idx_s, idx_d, rows, accum, gsem):
    c = lax.axis_index("c")
    s = lax.axis_index("s")
    tid = s * 2 + c
    nchunks = n_edges // _EC
    rpt = n_nodes // 16

    zero16 = jnp.zeros((16,), jnp.float32)

    def zrow(i, carry):
        rows[i, pl.ds(0, 16)] = zero16
        rows[i, pl.ds(16, 16)] = zero16
        return carry

    lax.fori_loop(0, _EC, zrow, 0)

    base = s * rpt
    nfull = rpt // _EC
    rem = rpt - nfull * _EC

    def zcopy(i, carry):
        pltpu.sync_copy(rows, accum.at[pl.ds(base + i * _EC, _EC)])
        return carry

    lax.fori_loop(0, nfull, zcopy, 0)
    if rem:
        pltpu.sync_copy(rows.at[pl.ds(0, rem)],
                        accum.at[pl.ds(base + nfull * _EC, rem)])

    if not gather_rows:
        one16 = jnp.ones((16,), jnp.float32)

        def orow(i, carry):
            rows[i, pl.ds(0, 16)] = one16
            rows[i, pl.ds(16, 16)] = one16
            return carry

        lax.fori_loop(0, _EC, orow, 0)

    plsc.subcore_barrier()

    def ebody(i, carry):
        j = tid + _NTILES * i

        @pl.when(j < nchunks)
        def _():
            pltpu.sync_copy(dst.at[pl.ds(j * _EC, _EC)], idx_d.at[0])
            if gather_rows:
                pltpu.sync_copy(src.at[pl.ds(j * _EC, _EC)], idx_s.at[0])
                pltpu.async_copy(vals.at[idx_s.at[0]], rows, gsem).wait()
            pltpu.sync_copy(rows, accum.at[idx_d.at[0]], add=True)

        return carry

    lax.fori_loop(0, (nchunks + _NTILES - 1) // _NTILES, ebody, 0)

    plsc.subcore_barrier()
    pltpu.sync_copy(accum.at[pl.ds(base, rpt)], out.at[c, pl.ds(base, rpt)])


def _sc_scatter_add(vals32, src, dst, gather_rows=True):
    n = vals32.shape[0]
    e = src.shape[0]
    npad = ((n + 127) // 128) * 128
    f = pl.kernel(
        functools.partial(_scatter_body, npad, e, gather_rows),
        out_type=jax.ShapeDtypeStruct((2, npad, _W), jnp.float32),
        mesh=plsc.VectorSubcoreMesh(core_axis_name="c", subcore_axis_name="s"),
        scratch_types=[
            pltpu.VMEM((1, _EC), jnp.int32),
            pltpu.VMEM((1, _EC), jnp.int32),
            pltpu.VMEM((_EC, _W), jnp.float32),
            pltpu.VMEM_SHARED((npad, _W), jnp.float32),
            pltpu.SemaphoreType.DMA,
        ],
        compiler_params=pltpu.CompilerParams(
            use_tc_tiling_on_sc=False, has_side_effects=True),
    )
    parts = f(vals32, src, dst)
    return parts[0, :n] + parts[1, :n]


def _permute(v, idx):
    dnums = lax.GatherDimensionNumbers(
        offset_dims=(), collapsed_slice_dims=(0,), start_index_map=(0,))
    return lax.gather(v, idx[:, None], dnums, slice_sizes=(1,),
                      mode=lax.GatherScatterMode.PROMISE_IN_BOUNDS)


def _segmax_body(n_nodes, nf, x, ids, out, ids_v, rows_v, buf_v, gsem):
    c = lax.axis_index("c")
    s = lax.axis_index("s")
    t = s * 2 + c
    neg = jnp.full((16,), -jnp.inf, dtype=jnp.float32)
    iota = lax.iota(jnp.int32, 16)

    def zb(i, carry):
        for j in range(nf // 16):
            buf_v[i, pl.ds(16 * j, 16)] = neg
        return carry

    lax.fori_loop(0, _G, zb, 0)

    nch_total = (n_nodes + _EC - 1) // _EC
    last_sz = n_nodes - (nch_total - 1) * _EC

    def chunk(i, carry):
        ci = t + _NTILES * i

        @pl.when(ci < nch_total)
        def _():
            @pl.when(ci < nch_total - 1)
            def _():
                pltpu.sync_copy(x.at[pl.ds(ci * _EC, _EC)], rows_v)
                pltpu.sync_copy(ids.at[pl.ds(ci * _EC, _EC)], ids_v.at[0])

            @pl.when(ci == nch_total - 1)
            def _():
                pltpu.sync_copy(x.at[pl.ds(ci * _EC, last_sz)],
                                rows_v.at[pl.ds(0, last_sz)])
                pltpu.sync_copy(ids.at[pl.ds(ci * _EC, last_sz)],
                                ids_v.at[0, pl.ds(0, last_sz)])
                pad16 = jnp.full((16,), -jnp.inf, dtype=jnp.float32)
                lastid = plsc.load_gather(
                    ids_v, [jnp.zeros((16,), jnp.int32),
                            jnp.full((16,), last_sz - 1, jnp.int32)])

                def padrow(r, carry2):
                    for j in range(nf // 16):
                        rows_v[r, pl.ds(16 * j, 16)] = pad16
                    return carry2

                lax.fori_loop(last_sz, _EC, padrow, 0)

                def padid(g, carry2):
                    base16 = g * 16
                    msk = (base16 + iota) >= last_sz
                    old = ids_v[0, pl.ds(base16, 16)]
                    ids_v[0, pl.ds(base16, 16)] = jnp.where(msk, lastid, old)
                    return carry2

                lax.fori_loop(last_sz // 16, _EC // 16, padid, 0)

            def grp(g, carry2):
                ids16 = ids_v[0, pl.ds(g * 16, 16)]
                row0 = g * 16
                sh = []
                for sft in (1, 2, 4, 8):
                    idx = jnp.maximum(iota - sft, 0)
                    eq = _permute(ids16, idx) == ids16
                    sh.append((idx, eq))
                nxt = jnp.minimum(iota + 1, 15)
                is_last = (iota == 15) | (_permute(ids16, nxt) != ids16)
                for j in range(nf):
                    jj = jnp.full((16,), j, jnp.int32)
                    m = plsc.load_gather(rows_v, [row0 + iota, jj])
                    for idx, eq in sh:
                        m = jnp.where(eq, jnp.maximum(m, _permute(m, idx)), m)
                    cur = plsc.load_gather(buf_v, [ids16, jj])
                    plsc.store_scatter(buf_v, [ids16, jj],
                                       jnp.maximum(cur, m), mask=is_last)
                return carry2

            lax.fori_loop(0, _EC // 16, grp, 0)

        return carry

    lax.fori_loop(0, (nch_total + _NTILES - 1) // _NTILES, chunk, 0)
    pltpu.sync_copy(buf_v, out.at[t])


def _sc_segment_max(x, ids):
    n, nf = x.shape
    f = pl.kernel(
        functools.partial(_segmax_body, n, nf),
        out_type=jax.ShapeDtypeStruct((_NTILES, _G, nf), jnp.float32),
        mesh=plsc.VectorSubcoreMesh(core_axis_name="c", subcore_axis_name="s"),
        scratch_types=[
            pltpu.VMEM((1, _EC), jnp.int32),
            pltpu.VMEM((_EC, nf), jnp.float32),
            pltpu.VMEM((_G, nf), jnp.float32),
            pltpu.SemaphoreType.DMA,
        ],
        compiler_params=pltpu.CompilerParams(
            use_tc_tiling_on_sc=False, needs_layout_passes=False,
            has_side_effects=True),
    )
    return f(x, ids)


def _dot(a, b):
    return lax.dot_general(a, b, (((1,), (0,)), ((), ())),
                           precision=lax.Precision.HIGHEST,
                           preferred_element_type=jnp.float32)


def _mm_body(x_ref, w_ref, o_ref):
    o_ref[...] = _dot(x_ref[...], w_ref[...])


def _tc_matmul(x, W):
    n, k = x.shape
    m = W.shape[1]
    return pl.pallas_call(
        _mm_body,
        grid=(n // _BR,),
        in_specs=[pl.BlockSpec((_BR, k), lambda i: (i, 0)),
                  pl.BlockSpec((k, m), lambda i: (0, 0))],
        out_specs=pl.BlockSpec((_BR, m), lambda i: (i, 0)),
        out_shape=jax.ShapeDtypeStruct((n, m), jnp.float32),
    )(x, W)


def _gin_post_body(y_ref, a_ref, ba_ref, wb_ref, bb_ref,
                   h_ref, s_ref):
    z = jnp.maximum(y_ref[...] + a_ref[...] + ba_ref[...], 0.0)
    h = jnp.maximum(_dot(z, wb_ref[...]) + bb_ref[...], 0.0)
    h_ref[...] = h
    ps = jnp.concatenate([jnp.sum(h, 0)[None], jnp.sum(h * h, 0)[None]], 0)

    @pl.when(pl.program_id(0) == 0)
    def _():
        s_ref[...] = ps

    @pl.when(pl.program_id(0) != 0)
    def _():
        s_ref[...] += ps


def _gin_post(y, agg, ba, Wb, bb):
    n, k = y.shape
    return pl.pallas_call(
        _gin_post_body,
        grid=(n // _BR,),
        in_specs=[
            pl.BlockSpec((_BR, k), lambda i: (i, 0)),
            pl.BlockSpec((_BR, k), lambda i: (i, 0)),
            pl.BlockSpec((1, k), lambda i: (0, 0)),
            pl.BlockSpec((k, k), lambda i: (0, 0)),
            pl.BlockSpec((1, k), lambda i: (0, 0)),
        ],
        out_specs=[
            pl.BlockSpec((_BR, k), lambda i: (i, 0)),
            pl.BlockSpec((2, k), lambda i: (0, 0)),
        ],
        out_shape=[
            jax.ShapeDtypeStruct((n, k), jnp.float32),
            jax.ShapeDtypeStruct((2, k), jnp.float32),
        ],
    )(y, agg, ba[None], Wb, bb[None])


def _bn_scale(s_ref, g_ref, b_ref, n):
    mu = s_ref[0] / n
    var = s_ref[1] / n - mu * mu
    sc = g_ref[0] * lax.rsqrt(var + 1e-5)
    t = b_ref[0] - mu * sc
    return sc, t


def _bn_mm_body(n, h_ref, s_ref, g_ref, b_ref, wa_ref, o_ref):
    sc, t = _bn_scale(s_ref, g_ref, b_ref, n)
    o_ref[...] = _dot(h_ref[...] * sc + t, wa_ref[...])


def _bn_mm(h, sums, gamma, beta, Wa):
    n, k = h.shape
    m = Wa.shape[1]
    return pl.pallas_call(
        functools.partial(_bn_mm_body, float(n)),
        grid=(n // _BR,),
        in_specs=[
            pl.BlockSpec((_BR, k), lambda i: (i, 0)),
            pl.BlockSpec((2, k), lambda i: (0, 0)),
            pl.BlockSpec((1, k), lambda i: (0, 0)),
            pl.BlockSpec((1, k), lambda i: (0, 0)),
            pl.BlockSpec((k, m), lambda i: (0, 0)),
        ],
        out_specs=pl.BlockSpec((_BR, m), lambda i: (i, 0)),
        out_shape=jax.ShapeDtypeStruct((n, m), jnp.float32),
    )(h, sums, gamma[None], beta[None], Wa)


def _bn_pool_body(n, h_ref, s_ref, g_ref, b_ref, ids_ref, o_ref):
    sc, t = _bn_scale(s_ref, g_ref, b_ref, n)
    x = h_ref[...] * sc + t
    ids = ids_ref[0, 0, :]
    oh = (ids[:, None] ==
          lax.broadcasted_iota(jnp.int32, (_BR, _G), 1)).astype(jnp.float32)
    contrib = lax.dot_general(oh, x, (((0,), (0,)), ((), ())),
                              precision=lax.Precision.HIGHEST,
                              preferred_element_type=jnp.float32)

    @pl.when(pl.program_id(0) == 0)
    def _():
        o_ref[...] = contrib

    @pl.when(pl.program_id(0) != 0)
    def _():
        o_ref[...] += contrib


def _bn_pool(h, sums, gamma, beta, ids):
    n, k = h.shape
    ids3 = ids.reshape(n // _BR, 1, _BR)
    return pl.pallas_call(
        functools.partial(_bn_pool_body, float(n)),
        grid=(n // _BR,),
        in_specs=[
            pl.BlockSpec((_BR, k), lambda i: (i, 0)),
            pl.BlockSpec((2, k), lambda i: (0, 0)),
            pl.BlockSpec((1, k), lambda i: (0, 0)),
            pl.BlockSpec((1, k), lambda i: (0, 0)),
            pl.BlockSpec((1, 1, _BR), lambda i: (i, 0, 0)),
        ],
        out_specs=pl.BlockSpec((_G, k), lambda i: (0, 0)),
        out_shape=jax.ShapeDtypeStruct((_G, k), jnp.float32),
    )(h, sums, gamma[None], beta[None], ids3)


def _prot_pre_body(x_ref, d_ref, v_ref, dv_ref):
    deg = d_ref[...][:, 0] + 1.0
    dinv = lax.rsqrt(deg)
    v_ref[...] = x_ref[...] * dinv[:, None]
    dv_ref[...] = jnp.broadcast_to(dinv[:, None], dv_ref.shape)


def _prot_pre(x_pad, deg_agg):
    n = x_pad.shape[0]
    return pl.pallas_call(
        _prot_pre_body,
        grid=(n // _BR,),
        in_specs=[
            pl.BlockSpec((_BR, _W), lambda i: (i, 0)),
            pl.BlockSpec((_BR, _W), lambda i: (i, 0)),
        ],
        out_specs=[
            pl.BlockSpec((_BR, _W), lambda i: (i, 0)),
            pl.BlockSpec((_BR, 8), lambda i: (i, 0)),
        ],
        out_shape=[
            jax.ShapeDtypeStruct((n, _W), jnp.float32),
            jax.ShapeDtypeStruct((n, 8), jnp.float32),
        ],
    )(x_pad, deg_agg)


def _prot_layer_body(mul_out, v_ref, a_ref, d_ref, w_ref, b_ref,
                     o_ref):
    dinv = d_ref[...][:, 0:1]
    w = dinv * (a_ref[...] + v_ref[...])
    o = jnp.maximum(_dot(w, w_ref[...]) + b_ref[...], 0.0)
    if mul_out:
        o = o * dinv
    o_ref[...] = o


def _prot_layer(vpad, agg, dinv8, Wp, bp, mul_out):
    n = vpad.shape[0]
    m = Wp.shape[1]
    return pl.pallas_call(
        functools.partial(_prot_layer_body, mul_out),
        grid=(n // _BR,),
        in_specs=[
            pl.BlockSpec((_BR, _W), lambda i: (i, 0)),
            pl.BlockSpec((_BR, _W), lambda i: (i, 0)),
            pl.BlockSpec((_BR, 8), lambda i: (i, 0)),
            pl.BlockSpec((_W, m), lambda i: (0, 0)),
            pl.BlockSpec((1, m), lambda i: (0, 0)),
        ],
        out_specs=pl.BlockSpec((_BR, m), lambda i: (i, 0)),
        out_shape=jax.ShapeDtypeStruct((n, m), jnp.float32),
    )(vpad, agg, dinv8, Wp, bp[None])


def _head_body(pd_ref, mx_ref, fxdW_ref, fxdb_ref, l1W_ref, l1b_ref,
               f1W_ref, f1b_ref, f2W_ref, f2b_ref, oW_ref, ob_ref, o_ref):
    m = mx_ref[...]
    m = jnp.where(jnp.isfinite(m), m, 0.0)
    xd = jnp.maximum(_dot(pd_ref[...], fxdW_ref[...]) + fxdb_ref[...], 0.0)
    xp = jnp.maximum(_dot(m, l1W_ref[...]) + l1b_ref[...], 0.0)
    xc = jnp.concatenate([xd, xp], axis=1)
    y = jnp.maximum(_dot(xc, f1W_ref[...]) + f1b_ref[...], 0.0)
    y = jnp.maximum(_dot(y, f2W_ref[...]) + f2b_ref[...], 0.0)
    o_ref[...] = _dot(y, oW_ref[...]) + ob_ref[...]


def _head(pooled, mx_parts, fxdW, fxdb, l1W, l1b, f1W, f1b, f2W, f2b, oW, ob):
    full = lambda a: pl.BlockSpec(a.shape, lambda: tuple(0 for _ in a.shape))
    args = (pooled, mx_parts, fxdW, fxdb[None], l1W, l1b[None],
            f1W, f1b[None], f2W, f2b[None], oW, ob[None])
    return pl.pallas_call(
        _head_body,
        in_specs=[full(a) for a in args],
        out_specs=pl.BlockSpec((_G, 1), lambda: (0, 0)),
        out_shape=jax.ShapeDtypeStruct((_G, 1), jnp.float32),
    )(*args)


def kernel(x_drug, edge_index_drug, batch_drug, x_prot, edge_index_prot, batch_prot, g1Wa, g1ba, g1Wb, g1bb, bn1g, bn1b, g2Wa, g2ba, g2Wb, g2bb, bn2g, bn2b, g3Wa, g3ba, g3Wb, g3bb, bn3g, bn3b, g4Wa, g4ba, g4Wb, g4bb, bn4g, bn4b, g5Wa, g5ba, g5Wb, g5bb, bn5g, bn5b, fxdW, fxdb, gc1W, gc1b, gc2W, gc2b, l1W, l1b, f1W, f1b, f2W, f2b, oW, ob):
    d = dict(locals())
    src_d, dst_d = edge_index_drug[0], edge_index_drug[1]
    src_p, dst_p = edge_index_prot[0], edge_index_prot[1]
    n_d = x_drug.shape[0]
    n_p = x_prot.shape[0]

    y = _tc_matmul(x_drug, g1Wa)
    agg = _sc_scatter_add(y, src_d, dst_d)
    deg_agg = _sc_scatter_add(
        jnp.zeros((n_p, _W), jnp.float32), src_p, dst_p, gather_rows=False)
    x_pad = jnp.pad(x_prot, ((0, 0), (0, _W - x_prot.shape[1])))
    vpad, dinv8 = _prot_pre(x_pad, deg_agg)

    h, sums = _gin_post(y, agg, g1ba, g1Wb, g1bb)
    y = _bn_mm(h, sums, bn1g, bn1b, g2Wa)
    agg = _sc_scatter_add(y, src_d, dst_d)

    W1p = jnp.zeros((_W, _W), jnp.float32).at[:25, :25].set(gc1W)
    b1p = jnp.pad(gc1b, (0, _W - gc1b.shape[0]))
    pagg = _sc_scatter_add(vpad, src_p, dst_p)
    v2 = _prot_layer(vpad, pagg, dinv8, W1p, b1p, mul_out=True)

    h, sums = _gin_post(y, agg, g2ba, g2Wb, g2bb)
    y = _bn_mm(h, sums, bn2g, bn2b, g3Wa)
    agg = _sc_scatter_add(y, src_d, dst_d)

    W2p = jnp.zeros((_W, 64), jnp.float32).at[:25, :].set(gc2W)
    pagg = _sc_scatter_add(v2, src_p, dst_p)
    xp2 = _prot_layer(v2, pagg, dinv8, W2p, gc2b, mul_out=False)

    h, sums = _gin_post(y, agg, g3ba, g3Wb, g3bb)
    y = _bn_mm(h, sums, bn3g, bn3b, g4Wa)
    agg = _sc_scatter_add(y, src_d, dst_d)

    mx_parts = _sc_segment_max(xp2, batch_prot)
    mx = jnp.max(mx_parts, axis=0)

    h, sums = _gin_post(y, agg, g4ba, g4Wb, g4bb)
    y = _bn_mm(h, sums, bn4g, bn4b, g5Wa)
    agg = _sc_scatter_add(y, src_d, dst_d)
    h, sums = _gin_post(y, agg, g5ba, g5Wb, g5bb)
    pooled = _bn_pool(h, sums, bn5g, bn5b, batch_drug)

    return _head(pooled, mx, fxdW, fxdb, l1W, l1b,
                 f1W, f1b, f2W, f2b, oW, ob)
